# baseline (speedup 1.0000x reference)
import functools

import jax
import jax.numpy as jnp
from jax import lax
from jax.experimental import pallas as pl
from jax.experimental.pallas import tpu as pltpu
from jax.experimental.pallas import tpu_sc as plsc

WORLD = 8
NC, NS = 2, 16
NW = NC * NS
SPAD = 64
BB = 8


def _make_sc_gather(R, V, D):
    bpw = R // NW
    npair = bpw // 2
    assert R % NW == 0 and bpw % 2 == 0

    mesh = plsc.VectorSubcoreMesh(core_axis_name="c", subcore_axis_name="s",
                                  num_cores=NC, num_subcores=NS)

    @functools.partial(
        pl.kernel,
        out_type=jax.ShapeDtypeStruct((SPAD * R, D), jnp.float32),
        mesh=mesh,
        scratch_types=[
            pltpu.VMEM((bpw, SPAD), jnp.int32),
            pltpu.VMEM((bpw, SPAD), jnp.int32),
            pltpu.VMEM((SPAD, D), jnp.float32),
            pltpu.VMEM((SPAD, D), jnp.float32),
            pltpu.SemaphoreType.DMA,
            pltpu.SemaphoreType.DMA,
            pltpu.SemaphoreType.DMA,
            pltpu.SemaphoreType.DMA,
        ],
    )
    def sc_call(ids_hbm, table_hbm, out_hbm,
                ids_v, dst_v, rows0, rows1, sem_g0, sem_g1, sem_s0, sem_s1):
        wid = lax.axis_index("s") * NC + lax.axis_index("c")
        b0 = wid * bpw

        pltpu.sync_copy(ids_hbm.at[wid], ids_v)

        def dbody(b, carry):
            for g in range(SPAD // 16):
                s16 = (jnp.int32(g * 16)
                       + lax.broadcasted_iota(jnp.int32, (16,), 0))
                dst_v[b, pl.ds(g * 16, 16)] = s16 * R + (b0 + b)
            return carry

        lax.fori_loop(0, bpw, dbody, 0)

        pltpu.async_copy(table_hbm.at[ids_v.at[0]], rows0, sem_g0)

        def pbody(p, carry):
            e = 2 * p
            o = e + 1

            @pl.when(p > 0)
            def _():
                pltpu.make_async_copy(rows1, out_hbm.at[dst_v.at[o]],
                                      sem_s1).wait()

            pltpu.async_copy(table_hbm.at[ids_v.at[o]], rows1, sem_g1)
            pltpu.make_async_copy(table_hbm.at[ids_v.at[e]], rows0,
                                  sem_g0).wait()
            pltpu.async_copy(rows0, out_hbm.at[dst_v.at[e]], sem_s0)
            pltpu.make_async_copy(rows0, out_hbm.at[dst_v.at[e]],
                                  sem_s0).wait()

            @pl.when(p < npair - 1)
            def _():
                pltpu.async_copy(table_hbm.at[ids_v.at[e + 2]], rows0, sem_g0)

            pltpu.make_async_copy(table_hbm.at[ids_v.at[o]], rows1,
                                  sem_g1).wait()
            pltpu.async_copy(rows1, out_hbm.at[dst_v.at[o]], sem_s1)
            return carry

        lax.fori_loop(0, npair, pbody, 0)
        pltpu.make_async_copy(rows1, out_hbm.at[dst_v.at[bpw - 1]],
                              sem_s1).wait()

    return sc_call


def _make_tc_finalize(R, S, V, D):
    local_vocab = V // WORLD

    def body(g_ref, ids_ref, o_ref):
        ids = ids_ref[...]
        shard = jnp.zeros((BB, S), jnp.int32)
        for r in range(1, WORLD):
            shard = shard + (ids >= r * local_vocab).astype(jnp.int32)
        shard_t = jnp.transpose(shard, (1, 0))
        rows = g_ref[pl.ds(0, S), :, :]
        for seg in range(WORLD):
            m = (shard_t == seg).astype(jnp.float32)[..., None]
            o_ref[:, :, seg * D:(seg + 1) * D] = rows * m

    return pl.pallas_call(
        body,
        out_shape=jax.ShapeDtypeStruct((S, R, WORLD * D), jnp.float32),
        grid=(R // BB,),
        in_specs=[
            pl.BlockSpec((SPAD, BB, D), lambda i: (0, i, 0)),
            pl.BlockSpec((BB, S), lambda i: (i, 0)),
        ],
        out_specs=pl.BlockSpec((S, BB, WORLD * D), lambda i: (0, i, 0)),
    )


def kernel(input_ids, weight):
    R, S = input_ids.shape
    V, D = weight.shape
    ids = input_ids.astype(jnp.int32)
    ids_pad = jnp.pad(ids, ((0, 0), (0, SPAD - S)))
    ids3 = ids_pad.reshape(NW, R // NW, SPAD)
    g = _make_sc_gather(R, V, D)(ids3, weight)
    g3 = g.reshape(SPAD, R, D)
    out_perm = _make_tc_finalize(R, S, V, D)(g3, ids)
    return jnp.transpose(out_perm, (1, 0, 2))


# speedup vs baseline: 1.8371x; 1.8371x over previous
import functools

import jax
import jax.numpy as jnp
from jax import lax
from jax.experimental import pallas as pl
from jax.experimental.pallas import tpu as pltpu
from jax.experimental.pallas import tpu_sc as plsc

WORLD = 8
NC, NS = 2, 16
NW = NC * NS
SPAD = 56
MS = 4
BB = 8


def _make_sc_gather(R, V, D):
    bpw = R // NW
    nchunk = SPAD // MS
    npair = nchunk // 2
    assert R % NW == 0 and nchunk % 2 == 0

    mesh = plsc.VectorSubcoreMesh(core_axis_name="c", subcore_axis_name="s",
                                  num_cores=NC, num_subcores=NS)

    @functools.partial(
        pl.kernel,
        out_type=jax.ShapeDtypeStruct((SPAD * R, D), jnp.float32),
        mesh=mesh,
        scratch_types=[
            pltpu.VMEM((nchunk, MS * bpw), jnp.int32),
            pltpu.VMEM((MS * bpw, D), jnp.float32),
            pltpu.VMEM((MS * bpw, D), jnp.float32),
            pltpu.SemaphoreType.DMA,
            pltpu.SemaphoreType.DMA,
            pltpu.SemaphoreType.DMA,
            pltpu.SemaphoreType.DMA,
        ],
    )
    def sc_call(ids_hbm, table_hbm, out_hbm,
                ids_v, rows0, rows1, sem_g0, sem_g1, sem_s0, sem_s1):
        wid = lax.axis_index("s") * NC + lax.axis_index("c")
        b0 = wid * bpw

        pltpu.sync_copy(ids_hbm.at[wid], ids_v)

        def put(rows, c, sem):
            for j in range(MS):
                pltpu.async_copy(
                    rows.at[pl.ds(j * bpw, bpw)],
                    out_hbm.at[pl.ds((c * MS + j) * R + b0, bpw)], sem)

        def drain(rows, c, sem):
            for j in range(MS):
                pltpu.make_async_copy(
                    rows.at[pl.ds(j * bpw, bpw)],
                    out_hbm.at[pl.ds((c * MS + j) * R + b0, bpw)], sem).wait()

        pltpu.async_copy(table_hbm.at[ids_v.at[0]], rows0, sem_g0)

        def pbody(p, carry):
            e = 2 * p
            o = e + 1

            @pl.when(p > 0)
            def _():
                drain(rows1, o, sem_s1)

            pltpu.async_copy(table_hbm.at[ids_v.at[o]], rows1, sem_g1)
            pltpu.make_async_copy(table_hbm.at[ids_v.at[e]], rows0,
                                  sem_g0).wait()
            put(rows0, e, sem_s0)
            drain(rows0, e, sem_s0)

            @pl.when(p < npair - 1)
            def _():
                pltpu.async_copy(table_hbm.at[ids_v.at[e + 2]], rows0, sem_g0)

            pltpu.make_async_copy(table_hbm.at[ids_v.at[o]], rows1,
                                  sem_g1).wait()
            put(rows1, o, sem_s1)
            return carry

        lax.fori_loop(0, npair, pbody, 0)
        drain(rows1, nchunk - 1, sem_s1)

    return sc_call


def _make_tc_finalize(R, S, V, D):
    local_vocab = V // WORLD

    def body(g_ref, ids_ref, o_ref):
        ids = ids_ref[...]
        shard = jnp.zeros((BB, S), jnp.int32)
        for r in range(1, WORLD):
            shard = shard + (ids >= r * local_vocab).astype(jnp.int32)
        shard_t = jnp.transpose(shard, (1, 0))
        rows = g_ref[pl.ds(0, S), :, :]
        for seg in range(WORLD):
            m = (shard_t == seg).astype(jnp.float32)[..., None]
            o_ref[:, :, seg * D:(seg + 1) * D] = rows * m

    return pl.pallas_call(
        body,
        out_shape=jax.ShapeDtypeStruct((S, R, WORLD * D), jnp.float32),
        grid=(R // BB,),
        in_specs=[
            pl.BlockSpec((SPAD, BB, D), lambda i: (0, i, 0)),
            pl.BlockSpec((BB, S), lambda i: (i, 0)),
        ],
        out_specs=pl.BlockSpec((S, BB, WORLD * D), lambda i: (0, i, 0)),
    )


def kernel(input_ids, weight):
    R, S = input_ids.shape
    V, D = weight.shape
    ids = input_ids.astype(jnp.int32)
    ids_pad = jnp.pad(ids, ((0, 0), (0, SPAD - S)))
    ids_t3 = jnp.transpose(ids_pad.reshape(NW, R // NW, SPAD), (0, 2, 1))
    ids_t3 = ids_t3.reshape(NW, SPAD // MS, MS * (R // NW))
    g = _make_sc_gather(R, V, D)(ids_t3, weight)
    g3 = g.reshape(SPAD, R, D)
    out_perm = _make_tc_finalize(R, S, V, D)(g3, ids)
    return jnp.transpose(out_perm, (1, 0, 2))


# speedup vs baseline: 2.9293x; 1.5946x over previous
import functools

import jax
import jax.numpy as jnp
from jax import lax
from jax.experimental import pallas as pl
from jax.experimental.pallas import tpu as pltpu
from jax.experimental.pallas import tpu_sc as plsc

WORLD = 8
NC, NS = 2, 16
NW = NC * NS
SPAD = 56
MS = 4
BB = 8
NBUF = 4


def _make_sc_gather(R, S, V, D):
    bpw = R // NW
    nchunk_all = SPAD // MS
    nchunk = -(-S // MS)
    assert R % NW == 0 and nchunk <= nchunk_all

    mesh = plsc.VectorSubcoreMesh(core_axis_name="c", subcore_axis_name="s",
                                  num_cores=NC, num_subcores=NS)

    @functools.partial(
        pl.kernel,
        out_type=jax.ShapeDtypeStruct((SPAD * R, D), jnp.float32),
        mesh=mesh,
        scratch_types=[
            pltpu.VMEM((nchunk_all, MS * bpw), jnp.int32),
            [pltpu.VMEM((MS * bpw, D), jnp.float32) for _ in range(NBUF)],
            [pltpu.SemaphoreType.DMA for _ in range(NBUF)],
            [pltpu.SemaphoreType.DMA for _ in range(NBUF)],
        ],
    )
    def sc_call(ids_hbm, table_hbm, out_hbm, ids_v, rows, sem_g, sem_w):
        wid = lax.axis_index("s") * NC + lax.axis_index("c")
        b0 = wid * bpw

        pltpu.sync_copy(ids_hbm.at[wid], ids_v)

        def gath(c):
            pltpu.async_copy(table_hbm.at[ids_v.at[c]], rows[c % NBUF],
                             sem_g[c % NBUF])

        def wait_gath(c):
            pltpu.make_async_copy(table_hbm.at[ids_v.at[c]], rows[c % NBUF],
                                  sem_g[c % NBUF]).wait()

        def put(c):
            for j in range(MS):
                pltpu.async_copy(
                    rows[c % NBUF].at[pl.ds(j * bpw, bpw)],
                    out_hbm.at[pl.ds((c * MS + j) * R + b0, bpw)],
                    sem_w[c % NBUF])

        def drain(c):
            for j in range(MS):
                pltpu.make_async_copy(
                    rows[c % NBUF].at[pl.ds(j * bpw, bpw)],
                    out_hbm.at[pl.ds((c * MS + j) * R + b0, bpw)],
                    sem_w[c % NBUF]).wait()

        for c in range(min(NBUF - 1, nchunk)):
            gath(c)
        for c in range(nchunk):
            wait_gath(c)
            put(c)
            if c > 0:
                drain(c - 1)
            if c + NBUF - 1 < nchunk:
                gath(c + NBUF - 1)
        drain(nchunk - 1)

    return sc_call


def _make_tc_finalize(R, S, V, D):
    local_vocab = V // WORLD

    def body(g_ref, ids_ref, o_ref):
        ids = ids_ref[...]
        shard = jnp.zeros((BB, S), jnp.int32)
        for r in range(1, WORLD):
            shard = shard + (ids >= r * local_vocab).astype(jnp.int32)
        shard_t = jnp.transpose(shard, (1, 0))
        rows = g_ref[pl.ds(0, S), :, :]
        for seg in range(WORLD):
            m = (shard_t == seg).astype(jnp.float32)[..., None]
            o_ref[:, :, seg * D:(seg + 1) * D] = rows * m

    return pl.pallas_call(
        body,
        out_shape=jax.ShapeDtypeStruct((S, R, WORLD * D), jnp.float32),
        grid=(R // BB,),
        in_specs=[
            pl.BlockSpec((SPAD, BB, D), lambda i: (0, i, 0)),
            pl.BlockSpec((BB, S), lambda i: (i, 0)),
        ],
        out_specs=pl.BlockSpec((S, BB, WORLD * D), lambda i: (0, i, 0)),
    )


def kernel(input_ids, weight):
    R, S = input_ids.shape
    V, D = weight.shape
    ids = input_ids.astype(jnp.int32)
    ids_pad = jnp.pad(ids, ((0, 0), (0, SPAD - S)))
    ids_t3 = jnp.transpose(ids_pad.reshape(NW, R // NW, SPAD), (0, 2, 1))
    ids_t3 = ids_t3.reshape(NW, SPAD // MS, MS * (R // NW))
    g = _make_sc_gather(R, S, V, D)(ids_t3, weight)
    g3 = g.reshape(SPAD, R, D)
    out_perm = _make_tc_finalize(R, S, V, D)(g3, ids)
    return jnp.transpose(out_perm, (1, 0, 2))


# speedup vs baseline: 6.8434x; 2.3362x over previous
import functools

import jax
import jax.numpy as jnp
from jax import lax
from jax.experimental import pallas as pl
from jax.experimental.pallas import tpu as pltpu
from jax.experimental.pallas import tpu_sc as plsc

WORLD = 8
NC, NS = 2, 16
NW = NC * NS
SPAD = 56
MS = 4
NBUF = 4
ZROWS = 256


def _make_sc_call(R, S, V, D):
    local_vocab = V // WORLD
    bpw = R // NW
    nchunk_all = SPAD // MS
    nfull = S // MS
    srem = S - nfull * MS
    half = srem * bpw
    rows_per_s = (R // 8) * 64
    assert bpw == ZROWS // 8 and srem > 0 and nfull + 1 <= nchunk_all

    mesh = plsc.VectorSubcoreMesh(core_axis_name="c", subcore_axis_name="s",
                                  num_cores=NC, num_subcores=NS)

    @functools.partial(
        pl.kernel,
        out_type=jax.ShapeDtypeStruct((R * S * WORLD, D), jnp.float32),
        mesh=mesh,
        scratch_types=[
            pltpu.VMEM((nchunk_all, MS * bpw), jnp.int32),
            pltpu.VMEM((nchunk_all, MS * bpw), jnp.int32),
            pltpu.VMEM((1, half), jnp.int32),
            pltpu.VMEM((1, half), jnp.int32),
            pltpu.VMEM((ZROWS, D), jnp.float32),
            [pltpu.VMEM((MS * bpw, D), jnp.float32) for _ in range(NBUF)],
            pltpu.SemaphoreType.DMA,
            [pltpu.SemaphoreType.DMA for _ in range(NBUF)],
            [pltpu.SemaphoreType.DMA for _ in range(NBUF)],
        ],
    )
    def sc_call(ids_hbm, table_hbm, zeros_hbm, out_hbm,
                ids_v, dst_v, ids_t, dst_t, zbuf, rows, sem_z, sem_g, sem_w):
        wid = lax.axis_index("s") * NC + lax.axis_index("c")
        b0 = wid * bpw

        pltpu.sync_copy(zeros_hbm, zbuf)
        zbase = (b0 // 8) * 64

        def zissue(s, carry):
            pltpu.async_copy(
                zbuf, out_hbm.at[pl.ds(s * rows_per_s + zbase, ZROWS)], sem_z)
            return carry

        lax.fori_loop(0, S, zissue, 0)

        pltpu.sync_copy(ids_hbm.at[wid], ids_v)

        lane = lax.broadcasted_iota(jnp.int32, (16,), 0)
        base16 = (jnp.int32(b0 // 8) + lax.div(lane, jnp.int32(8))) * 64 \
            + lax.rem(lane, jnp.int32(8))

        def dbody(c, carry):
            for g in range(MS * bpw // 16):
                v = ids_v[c, pl.ds(g * 16, 16)]
                seg = lax.div(v, jnp.int32(local_vocab))
                s = c * MS + g // 2
                base = base16 + jnp.int32((g % 2) * 2 * 64)
                dst_v[c, pl.ds(g * 16, 16)] = (
                    s * rows_per_s + base + seg * WORLD)
            return carry

        lax.fori_loop(0, nfull, dbody, 0)

        for g in range(half // 16):
            v = ids_v[nfull, pl.ds(g * 16, 16)]
            ids_t[0, pl.ds(g * 16, 16)] = v
            seg = lax.div(v, jnp.int32(local_vocab))
            s = nfull * MS + g // 2
            base = base16 + jnp.int32((g % 2) * 2 * 64)
            dst_t[0, pl.ds(g * 16, 16)] = s * rows_per_s + base + seg * WORLD

        def gath(c):
            pltpu.async_copy(table_hbm.at[ids_v.at[c]], rows[c % NBUF],
                             sem_g[c % NBUF])

        def wait_gath(c):
            pltpu.make_async_copy(table_hbm.at[ids_v.at[c]], rows[c % NBUF],
                                  sem_g[c % NBUF]).wait()

        def put(c):
            pltpu.async_copy(rows[c % NBUF], out_hbm.at[dst_v.at[c]],
                             sem_w[c % NBUF])

        def drain(c):
            pltpu.make_async_copy(rows[c % NBUF], out_hbm.at[dst_v.at[c]],
                                  sem_w[c % NBUF]).wait()

        for c in range(min(NBUF - 1, nfull)):
            gath(c)

        def zdrain(s, carry):
            pltpu.make_async_copy(
                zbuf, out_hbm.at[pl.ds(s * rows_per_s + zbase, ZROWS)],
                sem_z).wait()
            return carry

        lax.fori_loop(0, S, zdrain, 0)

        for c in range(nfull):
            wait_gath(c)
            put(c)
            if c > 0:
                drain(c - 1)
            if c + NBUF - 1 < nfull:
                gath(c + NBUF - 1)
        drain(nfull - 1)

        tail = rows[0].at[pl.ds(0, half)]
        pltpu.async_copy(table_hbm.at[ids_t.at[0]], tail, sem_g[0]).wait()
        pltpu.async_copy(tail, out_hbm.at[dst_t.at[0]], sem_w[0]).wait()

    return sc_call


def kernel(input_ids, weight):
    R, S = input_ids.shape
    V, D = weight.shape
    ids = input_ids.astype(jnp.int32)
    ids_pad = jnp.pad(ids, ((0, 0), (0, SPAD - S)))
    ids_t3 = jnp.transpose(ids_pad.reshape(NW, R // NW, SPAD), (0, 2, 1))
    ids_t3 = ids_t3.reshape(NW, SPAD // MS, MS * (R // NW))
    zeros = jnp.zeros((ZROWS, D), jnp.float32)
    flat = _make_sc_call(R, S, V, D)(ids_t3, weight, zeros)
    t = flat.reshape(S, R // 8, WORLD, 8, D)
    return t.transpose(1, 3, 0, 2, 4).reshape(R, S, WORLD * D)
